# Initial kernel scaffold; baseline (speedup 1.0000x reference)
import jax
import jax.numpy as jnp
from jax.experimental import pallas as pl


def kernel(coords, instance_gt, W):
    raise NotImplementedError("write your pallas kernel here")


# speedup vs baseline: 1.8595x; 1.8595x over previous
import functools

import jax
import jax.numpy as jnp
from jax import lax
from jax.experimental import pallas as pl
from jax.experimental.pallas import tpu as pltpu

_S = 64
_LANES = _S * _S


def _shift_rows(v, d, f):
    z = jnp.zeros((f, _LANES), v.dtype)
    if d > 0:
        return jnp.concatenate([v[f:, :], z], axis=0)
    return jnp.concatenate([z, v[:-f, :]], axis=0)


def _shift_y(v, d, f):
    w = f * _S
    z = jnp.zeros((_S, w), v.dtype)
    if d > 0:
        return jnp.concatenate([v[:, w:], z], axis=1)
    return jnp.concatenate([z, v[:, :-w]], axis=1)


def _shift_z(v, d, f, zi):
    z = jnp.zeros((_S, f), v.dtype)
    if d > 0:
        sh = jnp.concatenate([v[:, f:], z], axis=1)
        return jnp.where(zi < _S - f, sh, 0.0)
    sh = jnp.concatenate([z, v[:, :-f]], axis=1)
    return jnp.where(zi >= f, sh, 0.0)


def _pool2(m, f):
    t = m + _shift_rows(m, 1, f)
    t = t + _shift_y(t, 1, f)
    t = t + _shift_z_nowrap(t, f)
    return t


def _shift_z_nowrap(v, f):
    z = jnp.zeros((_S, f), v.dtype)
    return jnp.concatenate([v[:, f:], z], axis=1)


def _dense_body(vp_ref, cp_ref, o0, o1, o2, o3):
    sumv = vp_ref[0] + vp_ref[1]
    cnt = cp_ref[0] + cp_ref[1]
    xi = lax.broadcasted_iota(jnp.int32, (_S, _LANES), 0)
    li = lax.broadcasted_iota(jnp.int32, (_S, _LANES), 1)
    yi = li // _S
    zi = li % _S

    sa = jnp.where(cnt > 0, sumv / jnp.maximum(cnt, 1.0), 0.0)
    sc = (cnt > 0).astype(jnp.float32)

    for sidx, o_ref in enumerate((o0, o1, o2, o3)):
        f = 1 << sidx
        if f == 1:
            anchor = None
            actb = sc > 0
        else:
            anchor = ((xi % f) == 0) & ((yi % f) == 0) & ((zi % f) == 0)
            actb = anchor & (sc > 0)
        val = jnp.where(actb, sa / jnp.maximum(sc, 1e-12), 0.0)
        shs = (
            _shift_rows(val, -1, f),
            _shift_rows(val, 1, f),
            _shift_y(val, -1, f),
            _shift_y(val, 1, f),
            _shift_z(val, -1, f, zi),
            _shift_z(val, 1, f, zi),
        )
        for i, sh in enumerate(shs):
            sh_b = sh.astype(jnp.bfloat16).astype(jnp.float32)
            o_ref[i] = (actb & (jnp.abs(sh_b - val) < 0.01)).astype(jnp.int8)
            o_ref[6 + i] = (actb & (sh_b > 0)).astype(jnp.int8)
        if sidx < 3:
            sa = _pool2(sa, f)
            sc = _pool2(sc, f)


_dense_call = pl.pallas_call(
    _dense_body,
    out_shape=tuple(
        jax.ShapeDtypeStruct((12, _S, _LANES), jnp.int8) for _ in range(4)
    ),
)


def _scatter_xla(coords, instance_gt):
    lin = (coords[:, 0] * _S + coords[:, 1]) * _S + coords[:, 2]
    valf = instance_gt.astype(jnp.float32) + 1.0
    sumv = jnp.zeros((_S * _S * _S,), jnp.float32).at[lin].add(valf)
    cnt = jnp.zeros((_S * _S * _S,), jnp.float32).at[lin].add(1.0)
    zero = jnp.zeros((_S * _S * _S,), jnp.float32)
    vp = jnp.stack([sumv, zero]).reshape(2, _S, _LANES)
    cp = jnp.stack([cnt, zero]).reshape(2, _S, _LANES)
    return vp, cp


def kernel(coords, instance_gt, W):
    del W
    coords = coords.astype(jnp.int32)
    vp, cp = _scatter_xla(coords, instance_gt)
    embs = _dense_call(vp, cp)
    outs = []
    for sidx, emb in enumerate(embs):
        f = 1 << sidx
        g = emb.reshape(12, _S, _S, _S)[:, ::f, ::f, ::f]
        outs.append(g.transpose(1, 2, 3, 0).astype(jnp.float32))
    return tuple(outs)


# speedup vs baseline: 3.7940x; 2.0403x over previous
import functools

import jax
import jax.numpy as jnp
from jax import lax
from jax.experimental import pallas as pl
from jax.experimental.pallas import tpu as pltpu
from jax.experimental.pallas import tpu_sc as plsc

_S = 64
_LANES = _S * _S
_NVOX = _S * _S * _S
_NC = 2
_NS = 16
_NW = _NC * _NS
_STRIPE = _NVOX // _NS


def _shift_rows(v, d, f):
    z = jnp.zeros((f, _LANES), v.dtype)
    if d > 0:
        return jnp.concatenate([v[f:, :], z], axis=0)
    return jnp.concatenate([z, v[:-f, :]], axis=0)


def _shift_y(v, d, f):
    w = f * _S
    z = jnp.zeros((_S, w), v.dtype)
    if d > 0:
        return jnp.concatenate([v[:, w:], z], axis=1)
    return jnp.concatenate([z, v[:, :-w]], axis=1)


def _shift_z(v, d, f, zi):
    z = jnp.zeros((_S, f), v.dtype)
    if d > 0:
        sh = jnp.concatenate([v[:, f:], z], axis=1)
        return jnp.where(zi < _S - f, sh, 0.0)
    sh = jnp.concatenate([z, v[:, :-f]], axis=1)
    return jnp.where(zi >= f, sh, 0.0)


def _pool2(m, f):
    t = m + _shift_rows(m, 1, f)
    t = t + _shift_y(t, 1, f)
    t = t + _shift_z_nowrap(t, f)
    return t


def _shift_z_nowrap(v, f):
    z = jnp.zeros((_S, f), v.dtype)
    return jnp.concatenate([v[:, f:], z], axis=1)


def _dense_body(vp_ref, cp_ref, o0, o1, o2, o3):
    sumv = vp_ref[0] + vp_ref[1]
    cnt = cp_ref[0] + cp_ref[1]
    xi = lax.broadcasted_iota(jnp.int32, (_S, _LANES), 0)
    li = lax.broadcasted_iota(jnp.int32, (_S, _LANES), 1)
    yi = li // _S
    zi = li % _S

    sa = jnp.where(cnt > 0, sumv / jnp.maximum(cnt, 1.0), 0.0)
    sc = (cnt > 0).astype(jnp.float32)

    for sidx, o_ref in enumerate((o0, o1, o2, o3)):
        f = 1 << sidx
        if f == 1:
            anchor = None
            actb = sc > 0
        else:
            anchor = ((xi % f) == 0) & ((yi % f) == 0) & ((zi % f) == 0)
            actb = anchor & (sc > 0)
        val = jnp.where(actb, sa / jnp.maximum(sc, 1e-12), 0.0)
        shs = (
            _shift_rows(val, -1, f),
            _shift_rows(val, 1, f),
            _shift_y(val, -1, f),
            _shift_y(val, 1, f),
            _shift_z(val, -1, f, zi),
            _shift_z(val, 1, f, zi),
        )
        for i, sh in enumerate(shs):
            sh_b = sh.astype(jnp.bfloat16).astype(jnp.float32)
            o_ref[i] = (actb & (jnp.abs(sh_b - val) < 0.01)).astype(jnp.int8)
            o_ref[6 + i] = (actb & (sh_b > 0)).astype(jnp.int8)
        if sidx < 3:
            sa = _pool2(sa, f)
            sc = _pool2(sc, f)


_dense_call = pl.pallas_call(
    _dense_body,
    out_shape=tuple(
        jax.ShapeDtypeStruct((12, _S, _LANES), jnp.int8) for _ in range(4)
    ),
)


@functools.lru_cache(maxsize=None)
def _make_scatter(n_real: int):
    chunk = -(-n_real // _NW // 128) * 128
    n_pad = chunk * _NW
    n_rows = chunk // 128
    mesh = plsc.VectorSubcoreMesh(core_axis_name="c", subcore_axis_name="s")

    @functools.partial(
        pl.kernel,
        out_type=(
            jax.ShapeDtypeStruct((_NC * _NVOX,), jnp.float32),
            jax.ShapeDtypeStruct((_NC * _NVOX,), jnp.float32),
        ),
        mesh=mesh,
        scratch_types=[
            pltpu.VMEM((chunk,), jnp.int32),
            pltpu.VMEM((chunk,), jnp.int32),
            pltpu.VMEM((chunk,), jnp.int32),
            pltpu.VMEM((chunk,), jnp.int32),
            pltpu.VMEM((n_rows, 128), jnp.int32),
            pltpu.VMEM((n_rows, 128), jnp.float32),
            pltpu.VMEM((n_rows, 128), jnp.float32),
            pltpu.VMEM((_STRIPE,), jnp.float32),
            pltpu.VMEM_SHARED((_NVOX,), jnp.float32),
            pltpu.VMEM_SHARED((_NVOX,), jnp.float32),
            pltpu.SemaphoreType.DMA,
        ],
    )
    def scatter(x_hbm, y_hbm, z_hbm, gt_hbm, vout, cout, x_v, y_v, z_v,
                gt_v, idx_v, val_v, one_v, stage_v, gv, gc, sem):
        c = lax.axis_index("c")
        s = lax.axis_index("s")
        wid = s * _NC + c
        base = wid * chunk
        pltpu.sync_copy(x_hbm.at[pl.ds(base, chunk)], x_v)
        pltpu.sync_copy(y_hbm.at[pl.ds(base, chunk)], y_v)
        pltpu.sync_copy(z_hbm.at[pl.ds(base, chunk)], z_v)
        pltpu.sync_copy(gt_hbm.at[pl.ds(base, chunk)], gt_v)

        zeros16 = jnp.zeros((16,), jnp.float32)

        def zbody(i, _):
            stage_v[pl.ds(i * 16, 16)] = zeros16
            return 0

        lax.fori_loop(0, _STRIPE // 16, zbody, 0)
        pltpu.sync_copy(stage_v, gv.at[pl.ds(s * _STRIPE, _STRIPE)])
        pltpu.sync_copy(stage_v, gc.at[pl.ds(s * _STRIPE, _STRIPE)])

        iota16 = lax.iota(jnp.int32, 16)

        def cbody(i, _):
            rows = i * 16 + iota16
            x = x_v[pl.ds(i * 16, 16)]
            y = y_v[pl.ds(i * 16, 16)]
            z = z_v[pl.ds(i * 16, 16)]
            g = gt_v[pl.ds(i * 16, 16)]
            lin = (x * _S + y) * _S + z
            ok = (base + rows) < n_real
            j = i // 8
            col = (i % 8) * 16
            idx_v[j, pl.ds(col, 16)] = jnp.where(ok, lin, 0)
            val_v[j, pl.ds(col, 16)] = jnp.where(
                ok, g.astype(jnp.float32) + 1.0, 0.0)
            one_v[j, pl.ds(col, 16)] = jnp.where(ok, 1.0, 0.0)
            return 0

        lax.fori_loop(0, chunk // 16, cbody, 0)
        plsc.subcore_barrier()

        copies = []
        for j in range(n_rows):
            copies.append(
                pltpu.async_copy(val_v.at[j], gv.at[idx_v.at[j]], sem, add=True))
            copies.append(
                pltpu.async_copy(one_v.at[j], gc.at[idx_v.at[j]], sem, add=True))
        for cp_ in copies:
            cp_.wait()
        plsc.subcore_barrier()

        pltpu.sync_copy(gv.at[pl.ds(s * _STRIPE, _STRIPE)],
                        vout.at[pl.ds(c * _NVOX + s * _STRIPE, _STRIPE)])
        pltpu.sync_copy(gc.at[pl.ds(s * _STRIPE, _STRIPE)],
                        cout.at[pl.ds(c * _NVOX + s * _STRIPE, _STRIPE)])

    return scatter, n_pad


def _scatter_sc(coords, instance_gt):
    n_real = coords.shape[0]
    scatter, n_pad = _make_scatter(n_real)
    coords_p = jnp.pad(coords, ((0, n_pad - n_real), (0, 0)))
    gt_p = jnp.pad(instance_gt.astype(jnp.int32), (0, n_pad - n_real))
    vflat, cflat = scatter(coords_p[:, 0], coords_p[:, 1], coords_p[:, 2], gt_p)
    return (vflat.reshape(_NC, _S, _LANES), cflat.reshape(_NC, _S, _LANES))


def kernel(coords, instance_gt, W):
    del W
    coords = coords.astype(jnp.int32)
    vp, cp = _scatter_sc(coords, instance_gt)
    embs = _dense_call(vp, cp)
    outs = []
    for sidx, emb in enumerate(embs):
        f = 1 << sidx
        g = emb.reshape(12, _S, _S, _S)[:, ::f, ::f, ::f]
        outs.append(g.transpose(1, 2, 3, 0).astype(jnp.float32))
    return tuple(outs)


# speedup vs baseline: 20.1675x; 5.3156x over previous
import functools

import jax
import jax.numpy as jnp
from jax import lax
from jax.experimental import pallas as pl
from jax.experimental.pallas import tpu as pltpu
from jax.experimental.pallas import tpu_sc as plsc

_S = 64
_LANES = _S * _S
_NVOX = _S * _S * _S
_NC = 2
_NS = 16
_NW = _NC * _NS
_STRIPE = _NVOX // _NS


def _shift_rows(v, d, f):
    z = jnp.zeros((f, _LANES), v.dtype)
    if d > 0:
        return jnp.concatenate([v[f:, :], z], axis=0)
    return jnp.concatenate([z, v[:-f, :]], axis=0)


def _shift_y(v, d, f):
    w = f * _S
    z = jnp.zeros((_S, w), v.dtype)
    if d > 0:
        return jnp.concatenate([v[:, w:], z], axis=1)
    return jnp.concatenate([z, v[:, :-w]], axis=1)


def _shift_z(v, d, f, zi):
    z = jnp.zeros((_S, f), v.dtype)
    if d > 0:
        sh = jnp.concatenate([v[:, f:], z], axis=1)
        return jnp.where(zi < _S - f, sh, 0.0)
    sh = jnp.concatenate([z, v[:, :-f]], axis=1)
    return jnp.where(zi >= f, sh, 0.0)


def _pool2(m, f):
    t = m + _shift_rows(m, 1, f)
    t = t + _shift_y(t, 1, f)
    t = t + _shift_z_nowrap(t, f)
    return t


def _shift_z_nowrap(v, f):
    z = jnp.zeros((_S, f), v.dtype)
    return jnp.concatenate([v[:, f:], z], axis=1)


def _dense_body(vp_ref, cp_ref, o0, o1, o2, o3):
    sumv = vp_ref[0] + vp_ref[1]
    cnt = cp_ref[0] + cp_ref[1]
    xi = lax.broadcasted_iota(jnp.int32, (_S, _LANES), 0)
    li = lax.broadcasted_iota(jnp.int32, (_S, _LANES), 1)
    yi = li // _S
    zi = li % _S

    sa = jnp.where(cnt > 0, sumv / jnp.maximum(cnt, 1.0), 0.0)
    sc = (cnt > 0).astype(jnp.float32)

    for sidx, o_ref in enumerate((o0, o1, o2, o3)):
        f = 1 << sidx
        if f == 1:
            anchor = None
            actb = sc > 0
        else:
            anchor = ((xi % f) == 0) & ((yi % f) == 0) & ((zi % f) == 0)
            actb = anchor & (sc > 0)
        val = jnp.where(actb, sa / jnp.maximum(sc, 1e-12), 0.0)
        shs = (
            _shift_rows(val, -1, f),
            _shift_rows(val, 1, f),
            _shift_y(val, -1, f),
            _shift_y(val, 1, f),
            _shift_z(val, -1, f, zi),
            _shift_z(val, 1, f, zi),
        )
        acc = jnp.zeros((_S, _LANES), jnp.int32)
        for i, sh in enumerate(shs):
            sh_b = sh.astype(jnp.bfloat16).astype(jnp.float32)
            acc |= jnp.where(actb & (jnp.abs(sh_b - val) < 0.01), 1 << i, 0)
            acc |= jnp.where(actb & (sh_b > 0), 1 << (6 + i), 0)
        o_ref[...] = acc
        if sidx < 3:
            sa = _pool2(sa, f)
            sc = _pool2(sc, f)


_dense_call = pl.pallas_call(
    _dense_body,
    out_shape=tuple(
        jax.ShapeDtypeStruct((_S, _LANES), jnp.int32) for _ in range(4)
    ),
)


@functools.lru_cache(maxsize=None)
def _make_scatter(n_real: int):
    chunk = -(-n_real // _NW // 1024) * 1024
    n_pad = chunk * _NW
    n_rows = chunk // 128
    mesh = plsc.VectorSubcoreMesh(core_axis_name="c", subcore_axis_name="s")

    @functools.partial(
        pl.kernel,
        out_type=(
            jax.ShapeDtypeStruct((_NC * _NVOX,), jnp.float32),
            jax.ShapeDtypeStruct((_NC * _NVOX,), jnp.float32),
        ),
        mesh=mesh,
        scratch_types=[
            pltpu.VMEM((n_rows, 128), jnp.int32),
            pltpu.VMEM((n_rows, 128), jnp.float32),
            pltpu.VMEM((n_rows, 128), jnp.float32),
            pltpu.VMEM_SHARED((_NVOX,), jnp.float32),
            pltpu.VMEM_SHARED((_NVOX,), jnp.float32),
            pltpu.SemaphoreType.DMA,
        ],
    )
    def scatter(lin_hbm, val_hbm, one_hbm, zero_hbm, vout, cout,
                idx_v, val_v, one_v, gv, gc, sem):
        c = lax.axis_index("c")
        s = lax.axis_index("s")
        wid = s * _NC + c
        row0 = wid * n_rows
        pltpu.sync_copy(lin_hbm.at[pl.ds(row0, n_rows)], idx_v)
        pltpu.sync_copy(val_hbm.at[pl.ds(row0, n_rows)], val_v)
        pltpu.sync_copy(one_hbm.at[pl.ds(row0, n_rows)], one_v)
        pltpu.sync_copy(zero_hbm, gv.at[pl.ds(s * _STRIPE, _STRIPE)])
        pltpu.sync_copy(zero_hbm, gc.at[pl.ds(s * _STRIPE, _STRIPE)])
        plsc.subcore_barrier()

        copies = []
        for j in range(n_rows):
            copies.append(
                pltpu.async_copy(val_v.at[j], gv.at[idx_v.at[j]], sem, add=True))
            copies.append(
                pltpu.async_copy(one_v.at[j], gc.at[idx_v.at[j]], sem, add=True))
        for cp_ in copies:
            cp_.wait()
        plsc.subcore_barrier()

        pltpu.sync_copy(gv.at[pl.ds(s * _STRIPE, _STRIPE)],
                        vout.at[pl.ds(c * _NVOX + s * _STRIPE, _STRIPE)])
        pltpu.sync_copy(gc.at[pl.ds(s * _STRIPE, _STRIPE)],
                        cout.at[pl.ds(c * _NVOX + s * _STRIPE, _STRIPE)])

    return scatter, n_pad


def _scatter_sc(coords, instance_gt):
    n_real = coords.shape[0]
    scatter, n_pad = _make_scatter(n_real)
    lin = (coords[:, 0] * _S + coords[:, 1]) * _S + coords[:, 2]
    valf = instance_gt.astype(jnp.float32) + 1.0
    pad_idx = jnp.arange(n_pad - n_real, dtype=jnp.int32) % _NVOX
    lin_p = jnp.concatenate([lin, pad_idx]).reshape(-1, 128)
    val_p = jnp.pad(valf, (0, n_pad - n_real)).reshape(-1, 128)
    one_p = (jnp.arange(n_pad) < n_real).astype(jnp.float32).reshape(-1, 128)
    zero = jnp.zeros((_STRIPE,), jnp.float32)
    vflat, cflat = scatter(lin_p, val_p, one_p, zero)
    return (vflat.reshape(_NC, _S, _LANES), cflat.reshape(_NC, _S, _LANES))


def kernel(coords, instance_gt, W):
    del W
    coords = coords.astype(jnp.int32)
    vp, cp = _scatter_sc(coords, instance_gt)
    pks = _dense_call(vp, cp)
    bits = jnp.arange(12, dtype=jnp.int32)
    outs = []
    for sidx, pk in enumerate(pks):
        f = 1 << sidx
        g = pk.reshape(_S, _S, _S)[::f, ::f, ::f]
        outs.append(((g[..., None] >> bits) & 1).astype(jnp.float32))
    return tuple(outs)


# speedup vs baseline: 20.1676x; 1.0000x over previous
import functools

import jax
import jax.numpy as jnp
from jax import lax
from jax.experimental import pallas as pl
from jax.experimental.pallas import tpu as pltpu
from jax.experimental.pallas import tpu_sc as plsc

_S = 64
_LANES = _S * _S
_NVOX = _S * _S * _S
_NC = 2
_NS = 16
_NW = _NC * _NS
_STRIPE = _NVOX // _NS


def _shift_rows(v, d, f):
    z = jnp.zeros((f, _LANES), v.dtype)
    if d > 0:
        return jnp.concatenate([v[f:, :], z], axis=0)
    return jnp.concatenate([z, v[:-f, :]], axis=0)


def _shift_y(v, d, f):
    w = f * _S
    z = jnp.zeros((_S, w), v.dtype)
    if d > 0:
        return jnp.concatenate([v[:, w:], z], axis=1)
    return jnp.concatenate([z, v[:, :-w]], axis=1)


def _shift_z(v, d, f, zi):
    z = jnp.zeros((_S, f), v.dtype)
    if d > 0:
        sh = jnp.concatenate([v[:, f:], z], axis=1)
        return jnp.where(zi < _S - f, sh, 0.0)
    sh = jnp.concatenate([z, v[:, :-f]], axis=1)
    return jnp.where(zi >= f, sh, 0.0)


def _pool2(m, f):
    t = m + _shift_rows(m, 1, f)
    t = t + _shift_y(t, 1, f)
    t = t + _shift_z_nowrap(t, f)
    return t


def _shift_z_nowrap(v, f):
    z = jnp.zeros((_S, f), v.dtype)
    return jnp.concatenate([v[:, f:], z], axis=1)


def _dense_body(vp_ref, cp_ref, o0, o1, o2, o3):
    sumv = vp_ref[0] + vp_ref[1]
    cnt = cp_ref[0] + cp_ref[1]
    xi = lax.broadcasted_iota(jnp.int32, (_S, _LANES), 0)
    li = lax.broadcasted_iota(jnp.int32, (_S, _LANES), 1)
    yi = li // _S
    zi = li % _S

    sa = jnp.where(cnt > 0, sumv / jnp.maximum(cnt, 1.0), 0.0)
    sc = (cnt > 0).astype(jnp.float32)

    for sidx, o_ref in enumerate((o0, o1, o2, o3)):
        f = 1 << sidx
        if f == 1:
            anchor = None
            actb = sc > 0
        else:
            anchor = ((xi % f) == 0) & ((yi % f) == 0) & ((zi % f) == 0)
            actb = anchor & (sc > 0)
        val = jnp.where(actb, sa / jnp.maximum(sc, 1e-12), 0.0)
        shs = (
            _shift_rows(val, -1, f),
            _shift_rows(val, 1, f),
            _shift_y(val, -1, f),
            _shift_y(val, 1, f),
            _shift_z(val, -1, f, zi),
            _shift_z(val, 1, f, zi),
        )
        acc = jnp.zeros((_S, _LANES), jnp.int32)
        for i, sh in enumerate(shs):
            sh_b = sh.astype(jnp.bfloat16).astype(jnp.float32)
            acc |= jnp.where(actb & (jnp.abs(sh_b - val) < 0.01), 1 << i, 0)
            acc |= jnp.where(actb & (sh_b > 0), 1 << (6 + i), 0)
        o_ref[...] = acc
        if sidx < 3:
            sa = _pool2(sa, f)
            sc = _pool2(sc, f)


_dense_call = pl.pallas_call(
    _dense_body,
    out_shape=tuple(
        jax.ShapeDtypeStruct((_S, _LANES), jnp.int32) for _ in range(4)
    ),
)


@functools.lru_cache(maxsize=None)
def _make_scatter(n_real: int):
    chunk = -(-n_real // _NW // 1024) * 1024
    n_pad = chunk * _NW
    n_rows = chunk // 128
    mesh = plsc.VectorSubcoreMesh(core_axis_name="c", subcore_axis_name="s")

    @functools.partial(
        pl.kernel,
        out_type=(
            jax.ShapeDtypeStruct((_NC * _NVOX,), jnp.float32),
            jax.ShapeDtypeStruct((_NC * _NVOX,), jnp.float32),
        ),
        mesh=mesh,
        scratch_types=[
            pltpu.VMEM((n_rows, 128), jnp.int32),
            pltpu.VMEM((n_rows, 128), jnp.float32),
            pltpu.VMEM((n_rows, 128), jnp.float32),
            pltpu.VMEM_SHARED((_NVOX,), jnp.float32),
            pltpu.VMEM_SHARED((_NVOX,), jnp.float32),
            pltpu.SemaphoreType.DMA,
        ],
    )
    def scatter(lin_hbm, val_hbm, one_hbm, zero_hbm, vout, cout,
                idx_v, val_v, one_v, gv, gc, sem):
        c = lax.axis_index("c")
        s = lax.axis_index("s")
        wid = s * _NC + c
        row0 = wid * n_rows
        pltpu.sync_copy(lin_hbm.at[pl.ds(row0, n_rows)], idx_v)
        pltpu.sync_copy(val_hbm.at[pl.ds(row0, n_rows)], val_v)
        pltpu.sync_copy(one_hbm.at[pl.ds(row0, n_rows)], one_v)
        pltpu.sync_copy(zero_hbm, gv.at[pl.ds(s * _STRIPE, _STRIPE)])
        pltpu.sync_copy(zero_hbm, gc.at[pl.ds(s * _STRIPE, _STRIPE)])
        plsc.subcore_barrier()

        def sbody(g, _):
            cps = []
            for k in range(8):
                j = g * 8 + k
                cps.append(pltpu.async_copy(
                    val_v.at[j], gv.at[idx_v.at[j]], sem, add=True))
                cps.append(pltpu.async_copy(
                    one_v.at[j], gc.at[idx_v.at[j]], sem, add=True))
            for cp_ in cps:
                cp_.wait()
            return 0

        lax.fori_loop(0, n_rows // 8, sbody, 0)
        plsc.subcore_barrier()

        pltpu.sync_copy(gv.at[pl.ds(s * _STRIPE, _STRIPE)],
                        vout.at[pl.ds(c * _NVOX + s * _STRIPE, _STRIPE)])
        pltpu.sync_copy(gc.at[pl.ds(s * _STRIPE, _STRIPE)],
                        cout.at[pl.ds(c * _NVOX + s * _STRIPE, _STRIPE)])

    return scatter, n_pad


def _scatter_sc(coords, instance_gt):
    n_real = coords.shape[0]
    scatter, n_pad = _make_scatter(n_real)
    lin = (coords[:, 0] * _S + coords[:, 1]) * _S + coords[:, 2]
    valf = instance_gt.astype(jnp.float32) + 1.0
    pad_idx = jnp.arange(n_pad - n_real, dtype=jnp.int32) % _NVOX
    lin_p = jnp.concatenate([lin, pad_idx]).reshape(-1, 128)
    val_p = jnp.pad(valf, (0, n_pad - n_real)).reshape(-1, 128)
    one_p = (jnp.arange(n_pad) < n_real).astype(jnp.float32).reshape(-1, 128)
    zero = jnp.zeros((_STRIPE,), jnp.float32)
    vflat, cflat = scatter(lin_p, val_p, one_p, zero)
    return (vflat.reshape(_NC, _S, _LANES), cflat.reshape(_NC, _S, _LANES))


def kernel(coords, instance_gt, W):
    del W
    coords = coords.astype(jnp.int32)
    vp, cp = _scatter_sc(coords, instance_gt)
    pks = _dense_call(vp, cp)
    bits = jnp.arange(12, dtype=jnp.int32)
    outs = []
    for sidx, pk in enumerate(pks):
        f = 1 << sidx
        g = pk.reshape(_S, _S, _S)[::f, ::f, ::f]
        outs.append(((g[..., None] >> bits) & 1).astype(jnp.float32))
    return tuple(outs)
